# baseline (speedup 1.0000x reference)
import functools

import jax
import jax.numpy as jnp
from jax import lax
from jax.experimental import pallas as pl
from jax.experimental.pallas import tpu as pltpu
from jax.experimental.pallas import tpu_sc as plsc

_NW = 32
_LANES = 16


def _iota16():
    return lax.iota(jnp.int32, _LANES)


def _pack_table(table_t, n_rows, dim):
    n_packed = n_rows // 2
    n_full = n_rows // 128
    tail = n_rows - n_full * 128
    per_w = (n_full + _NW - 1) // _NW
    mesh = plsc.VectorSubcoreMesh(core_axis_name="c", subcore_axis_name="s")

    @functools.partial(
        pl.kernel,
        out_type=jax.ShapeDtypeStruct((n_packed, 2 * dim), jnp.float32),
        mesh=mesh,
        scratch_types=[
            pltpu.VMEM((dim, 128), jnp.float32),
            pltpu.VMEM((dim, 2 * dim), jnp.float32),
            pltpu.VMEM((dim, tail), jnp.float32) if tail else None,
        ],
        compiler_params=pltpu.CompilerParams(use_tc_tiling_on_sc=True, needs_layout_passes=False),
    )
    def pack_kernel(tab_hbm, out_hbm, src_v, dst_v, tail_v):
        wid = lax.axis_index("s") * 2 + lax.axis_index("c")

        def transpose_block(width):
            @pl.loop(0, width // 2)
            def _(k):
                for cv in range(width // _LANES):
                    half = (cv * _LANES) // dim
                    row16 = (cv * _LANES) % dim + _iota16()
                    col16 = jnp.full((_LANES,), 2 * k + half, jnp.int32)
                    val = plsc.load_gather(src_v, [row16, col16])
                    dst_v[k, pl.ds(cv * _LANES, _LANES)] = val

        @pl.loop(0, per_w)
        def _(t):
            blk = t * _NW + wid

            @pl.when(blk < n_full)
            def _():
                origin = pl.multiple_of(blk * 128, 128)
                pltpu.sync_copy(tab_hbm.at[:, pl.ds(origin, 128)], src_v)
                transpose_block(128)
                dst = pl.multiple_of(blk * (128 // 2), 64)
                pltpu.sync_copy(dst_v, out_hbm.at[pl.ds(dst, dim)])

        if tail:
            @pl.when(wid == _NW - 1)
            def _():
                pltpu.sync_copy(tab_hbm.at[:, pl.ds(n_full * 128, tail)], tail_v)

                @pl.loop(0, tail // 2)
                def _(k):
                    for cv in range(2 * dim // _LANES):
                        half = (cv * _LANES) // dim
                        row16 = (cv * _LANES) % dim + _iota16()
                        col16 = jnp.full((_LANES,), 2 * k + half, jnp.int32)
                        val = plsc.load_gather(tail_v, [row16, col16])
                        dst_v[k, pl.ds(cv * _LANES, _LANES)] = val

                pltpu.sync_copy(
                    dst_v.at[pl.ds(0, tail // 2)],
                    out_hbm.at[pl.ds(n_full * (128 // 2), tail // 2)],
                )

    return pack_kernel(table_t)


def _gather_t(packed, idx_flat_t, b0_n, b1_n, dim):
    n_tb = b0_n // 128
    groups = b1_n * n_tb
    per_w = groups // _NW
    mesh = plsc.VectorSubcoreMesh(core_axis_name="c", subcore_axis_name="s")

    @functools.partial(
        pl.kernel,
        out_type=jax.ShapeDtypeStruct((b1_n, dim, b0_n), jnp.float32),
        mesh=mesh,
        scratch_types=[
            pltpu.VMEM((128,), jnp.int32),
            pltpu.VMEM((128,), jnp.int32),
            pltpu.VMEM((128,), jnp.int32),
            pltpu.VMEM((128, 2 * dim), jnp.float32),
            pltpu.VMEM((dim, 128), jnp.float32),
            pltpu.SemaphoreType.DMA,
        ],
        compiler_params=pltpu.CompilerParams(use_tc_tiling_on_sc=True, needs_layout_passes=False),
    )
    def gather_kernel(tab_hbm, idx_hbm, out_hbm, idx_v, row_v, par_v, rows_v, out_v, sem):
        wid = lax.axis_index("s") * 2 + lax.axis_index("c")

        @pl.loop(0, per_w)
        def _(t):
            g = t * _NW + wid
            b1 = g // n_tb
            tb = g % n_tb
            pltpu.sync_copy(
                idx_hbm.at[pl.ds(pl.multiple_of(b1 * b0_n + tb * 128, 128), 128)],
                idx_v,
            )
            for rv in range(8):
                sl = pl.ds(rv * _LANES, _LANES)
                iv = idx_v[sl]
                row_v[sl] = lax.shift_right_logical(iv, 1)
                par_v[sl] = lax.mul(lax.bitwise_and(iv, 1), dim)
            pltpu.async_copy(tab_hbm.at[row_v], rows_v, sem).wait()
            @pl.loop(0, dim)
            def _(f):
                for rv in range(8):
                    sl = pl.ds(rv * _LANES, _LANES)
                    row16 = rv * _LANES + _iota16()
                    col16 = par_v[sl] + f
                    val = plsc.load_gather(rows_v, [row16, col16])
                    out_v[f, sl] = val

            pltpu.sync_copy(
                out_v, out_hbm.at[b1, :, pl.ds(pl.multiple_of(tb * 128, 128), 128)]
            )

    return gather_kernel(packed, idx_flat_t)


def kernel(input, index):
    n_rows, dim = input.shape
    b0_n, b1_n = index.shape

    table_t = input.T
    idx_flat_t = index.T.reshape(b1_n * b0_n)

    packed = _pack_table(table_t, n_rows, dim)
    out_t = _gather_t(packed, idx_flat_t, b0_n, b1_n, dim)
    return out_t.transpose(2, 0, 1)


# speedup vs baseline: 2.6829x; 2.6829x over previous
import functools

import jax
import jax.numpy as jnp
from jax import lax
from jax.experimental import pallas as pl
from jax.experimental.pallas import tpu as pltpu
from jax.experimental.pallas import tpu_sc as plsc

_NW = 32
_LANES = 16


def _iota16():
    return lax.iota(jnp.int32, _LANES)


def _pack_table_tc(table_t, n_rows, dim):
    blk = 1024
    grid = (n_rows + blk - 1) // blk

    def body(i_ref, o_ref):
        o_ref[:, 0:dim] = i_ref[...].T

    return pl.pallas_call(
        body,
        grid=(grid,),
        in_specs=[pl.BlockSpec((dim, blk), lambda i: (0, i))],
        out_specs=pl.BlockSpec((blk, 128), lambda i: (i, 0)),
        out_shape=jax.ShapeDtypeStruct((n_rows, 128), jnp.float32),
    )(table_t)


def _gather_t(packed, idx_flat_t, b0_n, b1_n, dim):
    n_tb = b0_n // 128
    groups = b1_n * n_tb
    per_w = groups // _NW
    mesh = plsc.VectorSubcoreMesh(core_axis_name="c", subcore_axis_name="s")

    @functools.partial(
        pl.kernel,
        out_type=jax.ShapeDtypeStruct((b1_n, dim, b0_n), jnp.float32),
        mesh=mesh,
        scratch_types=[
            pltpu.VMEM((128,), jnp.int32),
            pltpu.VMEM((128,), jnp.int32),
            pltpu.VMEM((128, 128), jnp.float32),
            pltpu.VMEM((128, 128), jnp.float32),
            pltpu.VMEM((dim, 128), jnp.float32),
            pltpu.SemaphoreType.DMA,
            pltpu.SemaphoreType.DMA,
        ],
        compiler_params=pltpu.CompilerParams(
            use_tc_tiling_on_sc=True, needs_layout_passes=False
        ),
    )
    def gather_kernel(
        tab_hbm, idx_hbm, out_hbm, row_v0, row_v1, rows_v0, rows_v1, out_v, sem0, sem1
    ):
        wid = lax.axis_index("s") * 2 + lax.axis_index("c")
        row_vs = (row_v0, row_v1)
        rows_vs = (rows_v0, rows_v1)
        sems = (sem0, sem1)

        def seq_group(s):
            g = s * _NW + wid
            return g // n_tb, g % n_tb

        def start(s, b):
            b1, tb = seq_group(s)
            pltpu.sync_copy(
                idx_hbm.at[pl.ds(pl.multiple_of(b1 * b0_n + tb * 128, 128), 128)],
                row_vs[b],
            )
            pltpu.async_copy(tab_hbm.at[row_vs[b]], rows_vs[b], sems[b])

        def wait(b):
            pltpu.make_async_copy(tab_hbm.at[row_vs[b]], rows_vs[b], sems[b]).wait()

        start(0, 0)

        @pl.loop(0, per_w, step=2)
        def _(t):
            for b in range(2):
                s = t + b

                @pl.when(s + 1 < per_w)
                def _():
                    start(s + 1, 1 - b)

                wait(b)
                rows_b = rows_vs[b]
                row16s = [rv * _LANES + _iota16() for rv in range(8)]

                @plsc.parallel_loop(0, dim, unroll=4)
                def _(f):
                    f16 = jnp.full((_LANES,), f, jnp.int32)
                    for rv in range(8):
                        val = plsc.load_gather(rows_b, [row16s[rv], f16])
                        out_v[f, pl.ds(rv * _LANES, _LANES)] = val

                b1, tb = seq_group(s)
                pltpu.sync_copy(
                    out_v,
                    out_hbm.at[b1, :, pl.ds(pl.multiple_of(tb * 128, 128), 128)],
                )

    return gather_kernel(packed, idx_flat_t)


def kernel(input, index):
    n_rows, dim = input.shape
    b0_n, b1_n = index.shape

    table_t = input.T
    idx_flat_t = index.T.reshape(b1_n * b0_n)

    packed = _pack_table_tc(table_t, n_rows, dim)
    out_t = _gather_t(packed, idx_flat_t, b0_n, b1_n, dim)
    return out_t.transpose(2, 0, 1)


# speedup vs baseline: 3.3765x; 1.2585x over previous
import functools

import jax
import jax.numpy as jnp
from jax import lax
from jax.experimental import pallas as pl
from jax.experimental.pallas import tpu as pltpu
from jax.experimental.pallas import tpu_sc as plsc

_NW = 32
_LANES = 16
_BLK = 768


def _iota16():
    return lax.iota(jnp.int32, _LANES)


def _pack_table_tc(table_t, n_rows, dim):
    n_half = 651 * _BLK
    n_packed = n_rows - n_half
    grid = (n_packed + _BLK - 1) // _BLK

    def body(lo_ref, hi_ref, o_ref):
        o_ref[...] = jnp.concatenate([lo_ref[...].T, hi_ref[...].T], axis=1)

    return pl.pallas_call(
        body,
        grid=(grid,),
        in_specs=[
            pl.BlockSpec((dim, _BLK), lambda i: (0, i)),
            pl.BlockSpec((dim, _BLK), lambda i: (0, i + 651)),
        ],
        out_specs=pl.BlockSpec((_BLK, 2 * dim), lambda i: (i, 0)),
        out_shape=jax.ShapeDtypeStruct((n_packed, 2 * dim), jnp.float32),
    )(table_t, table_t)


def _gather_t(packed, idx_flat_t, n_half, b0_n, b1_n, dim):
    n_tb = b0_n // 128
    groups = b1_n * n_tb
    per_w = groups // _NW
    per_idx = per_w * 128
    mesh = plsc.VectorSubcoreMesh(core_axis_name="c", subcore_axis_name="s")

    @functools.partial(
        pl.kernel,
        out_type=jax.ShapeDtypeStruct((b1_n, dim, b0_n), jnp.float32),
        mesh=mesh,
        scratch_types=[
            pltpu.VMEM((per_idx,), jnp.int32),
            pltpu.VMEM((per_idx,), jnp.int32),
            pltpu.VMEM((128, 2 * dim), jnp.float32),
            pltpu.VMEM((128, 2 * dim), jnp.float32),
            pltpu.VMEM((dim, 128), jnp.float32),
            pltpu.SemaphoreType.DMA,
            pltpu.SemaphoreType.DMA,
        ],
        compiler_params=pltpu.CompilerParams(
            use_tc_tiling_on_sc=True, needs_layout_passes=False
        ),
    )
    def gather_kernel(
        tab_hbm, idx_hbm, out_hbm, row_all, par_all, rows_v0, rows_v1, out_v, sem0, sem1
    ):
        wid = lax.axis_index("s") * 2 + lax.axis_index("c")
        rows_vs = (rows_v0, rows_v1)
        sems = (sem0, sem1)

        base = pl.multiple_of(wid * per_idx, 128)
        pltpu.sync_copy(idx_hbm.at[pl.ds(base, per_idx)], row_all)

        @plsc.parallel_loop(0, per_idx // _LANES, unroll=4)
        def _(v):
            sl = pl.ds(v * _LANES, _LANES)
            iv = row_all[sl]
            hi = iv >= n_half
            row_all[sl] = jnp.where(hi, iv - n_half, iv)
            par_all[sl] = jnp.where(hi, dim, 0).astype(jnp.int32)

        def gather_slice(s):
            return row_all.at[pl.ds(pl.multiple_of(s * 128, 128), 128)]

        def start(s, b):
            pltpu.async_copy(tab_hbm.at[gather_slice(s)], rows_vs[b], sems[b])

        def wait(s, b):
            pltpu.make_async_copy(
                tab_hbm.at[gather_slice(s)], rows_vs[b], sems[b]
            ).wait()

        start(0, 0)

        @pl.loop(0, per_w, step=2)
        def _(t):
            for b in range(2):
                s = t + b

                @pl.when(s + 1 < per_w)
                def _():
                    start(s + 1, 1 - b)

                wait(s, b)
                rows_b = rows_vs[b]
                row16s = [rv * _LANES + _iota16() for rv in range(8)]
                par16s = [
                    par_all[pl.ds(s * 128 + rv * _LANES, _LANES)] for rv in range(8)
                ]

                @plsc.parallel_loop(0, dim, unroll=4)
                def _(f):
                    for rv in range(8):
                        val = plsc.load_gather(rows_b, [row16s[rv], par16s[rv] + f])
                        out_v[f, pl.ds(rv * _LANES, _LANES)] = val

                g = wid * per_w + s
                b1 = g // n_tb
                tb = g % n_tb
                pltpu.sync_copy(
                    out_v,
                    out_hbm.at[b1, :, pl.ds(pl.multiple_of(tb * 128, 128), 128)],
                )

    return gather_kernel(packed, idx_flat_t)


def kernel(input, index):
    n_rows, dim = input.shape
    b0_n, b1_n = index.shape
    n_half = 651 * _BLK

    table_t = input.T
    idx_flat_t = index.T.reshape(b1_n * b0_n)

    packed = _pack_table_tc(table_t, n_rows, dim)
    out_t = _gather_t(packed, idx_flat_t, n_half, b0_n, b1_n, dim)
    return out_t.transpose(2, 0, 1)


# speedup vs baseline: 4.6217x; 1.3688x over previous
import functools

import jax
import jax.numpy as jnp
from jax import lax
from jax.experimental import pallas as pl
from jax.experimental.pallas import tpu as pltpu
from jax.experimental.pallas import tpu_sc as plsc

_NW = 32
_LANES = 16
_BLK = 3072
_NHI = 162


def _iota16():
    return lax.iota(jnp.int32, _LANES)


def _pack_table_tc(table_t, n_rows, dim):
    n_half = _NHI * _BLK
    n_packed = n_rows - n_half
    grid = (n_packed + _BLK - 1) // _BLK

    def body(lo_ref, hi_ref, o_ref):
        o_ref[...] = jnp.concatenate([lo_ref[...].T, hi_ref[...].T], axis=1)

    return pl.pallas_call(
        body,
        grid=(grid,),
        in_specs=[
            pl.BlockSpec((dim, _BLK), lambda i: (0, i)),
            pl.BlockSpec((dim, _BLK), lambda i: (0, i + _NHI)),
        ],
        out_specs=pl.BlockSpec((_BLK, 2 * dim), lambda i: (i, 0)),
        out_shape=jax.ShapeDtypeStruct((n_packed, 2 * dim), jnp.float32),
    )(table_t, table_t)


def _gather_t(packed, idx_flat_t, n_half, b0_n, b1_n, dim):
    n_tb = b0_n // 128
    groups = b1_n * n_tb
    per_w = groups // _NW
    per_idx = per_w * 128
    mesh = plsc.VectorSubcoreMesh(core_axis_name="c", subcore_axis_name="s")

    @functools.partial(
        pl.kernel,
        out_type=jax.ShapeDtypeStruct((b1_n, dim, b0_n), jnp.float32),
        mesh=mesh,
        scratch_types=[
            pltpu.VMEM((per_idx,), jnp.int32),
            pltpu.VMEM((per_idx,), jnp.int32),
            pltpu.VMEM((128, 2 * dim), jnp.float32),
            pltpu.VMEM((128, 2 * dim), jnp.float32),
            pltpu.VMEM((dim, 128), jnp.float32),
            pltpu.SemaphoreType.DMA,
            pltpu.SemaphoreType.DMA,
        ],
        compiler_params=pltpu.CompilerParams(
            use_tc_tiling_on_sc=True, needs_layout_passes=False
        ),
    )
    def gather_kernel(
        tab_hbm, idx_hbm, out_hbm, row_all, par_all, rows_v0, rows_v1, out_v, sem0, sem1
    ):
        wid = lax.axis_index("s") * 2 + lax.axis_index("c")
        rows_vs = (rows_v0, rows_v1)
        sems = (sem0, sem1)

        base = pl.multiple_of(wid * per_idx, 128)
        pltpu.sync_copy(idx_hbm.at[pl.ds(base, per_idx)], row_all)

        @plsc.parallel_loop(0, per_idx // _LANES, unroll=4)
        def _(v):
            sl = pl.ds(v * _LANES, _LANES)
            iv = row_all[sl]
            hi = iv >= n_half
            row_all[sl] = jnp.where(hi, iv - n_half, iv)
            par_all[sl] = jnp.where(hi, dim, 0).astype(jnp.int32)

        def gather_slice(s):
            return row_all.at[pl.ds(pl.multiple_of(s * 128, 128), 128)]

        def start(s, b):
            pltpu.async_copy(tab_hbm.at[gather_slice(s)], rows_vs[b], sems[b])

        def wait(s, b):
            pltpu.make_async_copy(
                tab_hbm.at[gather_slice(s)], rows_vs[b], sems[b]
            ).wait()

        start(0, 0)

        @pl.loop(0, per_w, step=2)
        def _(t):
            for b in range(2):
                s = t + b

                @pl.when(s + 1 < per_w)
                def _():
                    start(s + 1, 1 - b)

                wait(s, b)
                rows_b = rows_vs[b]
                row16s = [rv * _LANES + _iota16() for rv in range(8)]
                par16s = [
                    par_all[pl.ds(s * 128 + rv * _LANES, _LANES)] for rv in range(8)
                ]

                @plsc.parallel_loop(0, dim, unroll=4)
                def _(f):
                    for rv in range(8):
                        val = plsc.load_gather(rows_b, [row16s[rv], par16s[rv] + f])
                        out_v[f, pl.ds(rv * _LANES, _LANES)] = val

                g = wid * per_w + s
                b1 = g // n_tb
                tb = g % n_tb
                pltpu.sync_copy(
                    out_v,
                    out_hbm.at[b1, :, pl.ds(pl.multiple_of(tb * 128, 128), 128)],
                )

    return gather_kernel(packed, idx_flat_t)


def kernel(input, index):
    n_rows, dim = input.shape
    b0_n, b1_n = index.shape
    n_half = _NHI * _BLK

    table_t = input.T
    idx_flat_t = index.T.reshape(b1_n * b0_n)

    packed = _pack_table_tc(table_t, n_rows, dim)
    out_t = _gather_t(packed, idx_flat_t, n_half, b0_n, b1_n, dim)
    return out_t.transpose(2, 0, 1)


# speedup vs baseline: 4.9512x; 1.0713x over previous
import functools

import jax
import jax.numpy as jnp
from jax import lax
from jax.experimental import pallas as pl
from jax.experimental.pallas import tpu as pltpu
from jax.experimental.pallas import tpu_sc as plsc

_NW = 32
_LANES = 16
_BLK = 6144
_NHI = 81


def _iota16():
    return lax.iota(jnp.int32, _LANES)


def _pack_table_tc(table_t, n_rows, dim):
    n_half = _NHI * _BLK
    n_packed = n_rows - n_half
    grid = (n_packed + _BLK - 1) // _BLK

    def body(lo_ref, hi_ref, o_ref):
        o_ref[...] = jnp.concatenate([lo_ref[...].T, hi_ref[...].T], axis=1)

    return pl.pallas_call(
        body,
        grid=(grid,),
        in_specs=[
            pl.BlockSpec((dim, _BLK), lambda i: (0, i)),
            pl.BlockSpec((dim, _BLK), lambda i: (0, i + _NHI)),
        ],
        out_specs=pl.BlockSpec((_BLK, 2 * dim), lambda i: (i, 0)),
        out_shape=jax.ShapeDtypeStruct((n_packed, 2 * dim), jnp.float32),
    )(table_t, table_t)


def _gather_t(packed, idx_flat_t, n_half, b0_n, b1_n, dim):
    n_tb = b0_n // 128
    groups = b1_n * n_tb
    per_w = groups // _NW
    per_idx = per_w * 128
    mesh = plsc.VectorSubcoreMesh(core_axis_name="c", subcore_axis_name="s")

    @functools.partial(
        pl.kernel,
        out_type=jax.ShapeDtypeStruct((b1_n, dim, b0_n), jnp.float32),
        mesh=mesh,
        scratch_types=[
            pltpu.VMEM((per_idx,), jnp.int32),
            pltpu.VMEM((per_idx,), jnp.int32),
            pltpu.VMEM((128, 2 * dim), jnp.float32),
            pltpu.VMEM((128, 2 * dim), jnp.float32),
            pltpu.VMEM((dim, 128), jnp.float32),
            pltpu.SemaphoreType.DMA,
            pltpu.SemaphoreType.DMA,
        ],
        compiler_params=pltpu.CompilerParams(
            use_tc_tiling_on_sc=True, needs_layout_passes=False
        ),
    )
    def gather_kernel(
        tab_hbm, idx_hbm, out_hbm, row_all, par_all, rows_v0, rows_v1, out_v, sem0, sem1
    ):
        wid = lax.axis_index("s") * 2 + lax.axis_index("c")
        rows_vs = (rows_v0, rows_v1)
        sems = (sem0, sem1)

        base = pl.multiple_of(wid * per_idx, 128)
        pltpu.sync_copy(idx_hbm.at[pl.ds(base, per_idx)], row_all)

        @plsc.parallel_loop(0, per_idx // _LANES, unroll=4)
        def _(v):
            sl = pl.ds(v * _LANES, _LANES)
            iv = row_all[sl]
            hi = iv >= n_half
            row_all[sl] = jnp.where(hi, iv - n_half, iv)
            par_all[sl] = jnp.where(hi, dim, 0).astype(jnp.int32)

        def gather_slice(s):
            return row_all.at[pl.ds(pl.multiple_of(s * 128, 128), 128)]

        def start(s, b):
            pltpu.async_copy(tab_hbm.at[gather_slice(s)], rows_vs[b], sems[b])

        def wait(s, b):
            pltpu.make_async_copy(
                tab_hbm.at[gather_slice(s)], rows_vs[b], sems[b]
            ).wait()

        start(0, 0)

        @pl.loop(0, per_w, step=2)
        def _(t):
            for b in range(2):
                s = t + b

                @pl.when(s + 1 < per_w)
                def _():
                    start(s + 1, 1 - b)

                wait(s, b)
                rows_b = rows_vs[b]
                row16s = [rv * _LANES + _iota16() for rv in range(8)]
                par16s = [
                    par_all[pl.ds(s * 128 + rv * _LANES, _LANES)] for rv in range(8)
                ]

                @plsc.parallel_loop(0, dim, unroll=8)
                def _(f):
                    for rv in range(8):
                        val = plsc.load_gather(rows_b, [row16s[rv], par16s[rv] + f])
                        out_v[f, pl.ds(rv * _LANES, _LANES)] = val

                g = wid * per_w + s
                b1 = g // n_tb
                tb = g % n_tb
                pltpu.sync_copy(
                    out_v,
                    out_hbm.at[b1, :, pl.ds(pl.multiple_of(tb * 128, 128), 128)],
                )

    return gather_kernel(packed, idx_flat_t)


def kernel(input, index):
    n_rows, dim = input.shape
    b0_n, b1_n = index.shape
    n_half = _NHI * _BLK

    table_t = input.T
    idx_flat_t = index.T.reshape(b1_n * b0_n)

    packed = _pack_table_tc(table_t, n_rows, dim)
    out_t = _gather_t(packed, idx_flat_t, n_half, b0_n, b1_n, dim)
    return out_t.transpose(2, 0, 1)


# speedup vs baseline: 5.2607x; 1.0625x over previous
import functools

import jax
import jax.numpy as jnp
from jax import lax
from jax.experimental import pallas as pl
from jax.experimental.pallas import tpu as pltpu
from jax.experimental.pallas import tpu_sc as plsc

_NW = 32
_LANES = 16
_BLK = 6144
_NHI = 81


def _iota16():
    return lax.iota(jnp.int32, _LANES)


def _pack_table_tc(table_t, n_rows, dim):
    n_half = _NHI * _BLK
    n_packed = n_rows - n_half
    grid = (n_packed + _BLK - 1) // _BLK

    def body(lo_ref, hi_ref, o_ref):
        o_ref[...] = jnp.concatenate([lo_ref[...].T, hi_ref[...].T], axis=1)

    return pl.pallas_call(
        body,
        grid=(grid,),
        in_specs=[
            pl.BlockSpec((dim, _BLK), lambda i: (0, i)),
            pl.BlockSpec((dim, _BLK), lambda i: (0, i + _NHI)),
        ],
        out_specs=pl.BlockSpec((_BLK, 2 * dim), lambda i: (i, 0)),
        out_shape=jax.ShapeDtypeStruct((n_packed, 2 * dim), jnp.float32),
    )(table_t, table_t)


def _gather_t(packed, idx_flat_t, n_half, b0_n, b1_n, dim):
    n_tb = b0_n // 128
    groups = b1_n * n_tb
    per_w = groups // _NW
    per_idx = per_w * 128
    mesh = plsc.VectorSubcoreMesh(core_axis_name="c", subcore_axis_name="s")

    @functools.partial(
        pl.kernel,
        out_type=jax.ShapeDtypeStruct((b1_n, dim, b0_n), jnp.float32),
        mesh=mesh,
        scratch_types=[
            pltpu.VMEM((per_idx,), jnp.int32),
            pltpu.VMEM((per_idx,), jnp.int32),
            pltpu.VMEM((128, 2 * dim), jnp.float32),
            pltpu.VMEM((128, 2 * dim), jnp.float32),
            pltpu.VMEM((dim, 128), jnp.float32),
            pltpu.VMEM((dim, 128), jnp.float32),
            pltpu.SemaphoreType.DMA,
            pltpu.SemaphoreType.DMA,
            pltpu.SemaphoreType.DMA,
            pltpu.SemaphoreType.DMA,
        ],
        compiler_params=pltpu.CompilerParams(
            use_tc_tiling_on_sc=True, needs_layout_passes=False
        ),
    )
    def gather_kernel(
        tab_hbm, idx_hbm, out_hbm, row_all, par_all, rows_v0, rows_v1,
        out_v0, out_v1, sem0, sem1, semo0, semo1
    ):
        wid = lax.axis_index("s") * 2 + lax.axis_index("c")
        rows_vs = (rows_v0, rows_v1)
        out_vs = (out_v0, out_v1)
        sems = (sem0, sem1)
        semos = (semo0, semo1)

        base = pl.multiple_of(wid * per_idx, 128)
        pltpu.sync_copy(idx_hbm.at[pl.ds(base, per_idx)], row_all)

        @plsc.parallel_loop(0, per_idx // _LANES, unroll=4)
        def _(v):
            sl = pl.ds(v * _LANES, _LANES)
            iv = row_all[sl]
            hi = iv >= n_half
            row_all[sl] = jnp.where(hi, iv - n_half, iv)
            par_all[sl] = jnp.where(hi, dim, 0).astype(jnp.int32)

        def gather_slice(s):
            return row_all.at[pl.ds(pl.multiple_of(s * 128, 128), 128)]

        def start(s, b):
            pltpu.async_copy(tab_hbm.at[gather_slice(s)], rows_vs[b], sems[b])

        def wait(s, b):
            pltpu.make_async_copy(
                tab_hbm.at[gather_slice(s)], rows_vs[b], sems[b]
            ).wait()

        def out_slice(s):
            g = wid * per_w + s
            b1 = g // n_tb
            tb = g % n_tb
            return out_hbm.at[b1, :, pl.ds(pl.multiple_of(tb * 128, 128), 128)]

        def wait_out(s, b):
            pltpu.make_async_copy(out_vs[b], out_slice(s), semos[b]).wait()

        start(0, 0)

        @pl.loop(0, per_w, step=2)
        def _(t):
            for b in range(2):
                s = t + b

                @pl.when(s + 1 < per_w)
                def _():
                    start(s + 1, 1 - b)

                wait(s, b)

                @pl.when(s >= 2)
                def _():
                    wait_out(s - 2, b)

                rows_b = rows_vs[b]
                out_b = out_vs[b]
                row16s = [rv * _LANES + _iota16() for rv in range(8)]
                par16s = [
                    par_all[pl.ds(s * 128 + rv * _LANES, _LANES)] for rv in range(8)
                ]

                @plsc.parallel_loop(0, dim, unroll=8)
                def _(f):
                    for rv in range(8):
                        val = plsc.load_gather(rows_b, [row16s[rv], par16s[rv] + f])
                        out_b[f, pl.ds(rv * _LANES, _LANES)] = val

                pltpu.async_copy(out_vs[b], out_slice(s), semos[b])

        wait_out(per_w - 2, 0)
        wait_out(per_w - 1, 1)

    return gather_kernel(packed, idx_flat_t)


def kernel(input, index):
    n_rows, dim = input.shape
    b0_n, b1_n = index.shape
    n_half = _NHI * _BLK

    table_t = input.T
    idx_flat_t = index.T.reshape(b1_n * b0_n)

    packed = _pack_table_tc(table_t, n_rows, dim)
    out_t = _gather_t(packed, idx_flat_t, n_half, b0_n, b1_n, dim)
    return out_t.transpose(2, 0, 1)


# speedup vs baseline: 5.4216x; 1.0306x over previous
import functools

import jax
import jax.numpy as jnp
from jax import lax
from jax.experimental import pallas as pl
from jax.experimental.pallas import tpu as pltpu
from jax.experimental.pallas import tpu_sc as plsc

_NW = 32
_LANES = 16
_BLK = 12288
_NHI = 40


def _iota16():
    return lax.iota(jnp.int32, _LANES)


def _pack_table_tc(table_t, n_rows, dim):
    n_half = _NHI * _BLK
    n_packed = n_rows - n_half
    grid = (n_packed + _BLK - 1) // _BLK

    def body(lo_ref, hi_ref, o_ref):
        o_ref[...] = jnp.concatenate([lo_ref[...].T, hi_ref[...].T], axis=1)

    return pl.pallas_call(
        body,
        grid=(grid,),
        in_specs=[
            pl.BlockSpec((dim, _BLK), lambda i: (0, i)),
            pl.BlockSpec((dim, _BLK), lambda i: (0, i + _NHI)),
        ],
        out_specs=pl.BlockSpec((_BLK, 2 * dim), lambda i: (i, 0)),
        out_shape=jax.ShapeDtypeStruct((n_packed, 2 * dim), jnp.float32),
    )(table_t, table_t)


def _gather_t(packed, idx_flat_t, n_half, b0_n, b1_n, dim):
    n_tb = b0_n // 128
    groups = b1_n * n_tb
    per_w = groups // _NW
    per_idx = per_w * 128
    mesh = plsc.VectorSubcoreMesh(core_axis_name="c", subcore_axis_name="s")

    @functools.partial(
        pl.kernel,
        out_type=jax.ShapeDtypeStruct((b1_n, dim, b0_n), jnp.float32),
        mesh=mesh,
        scratch_types=[
            pltpu.VMEM((per_idx,), jnp.int32),
            pltpu.VMEM((per_idx,), jnp.int32),
            pltpu.VMEM((128, 2 * dim), jnp.float32),
            pltpu.VMEM((128, 2 * dim), jnp.float32),
            pltpu.VMEM((dim, 128), jnp.float32),
            pltpu.VMEM((dim, 128), jnp.float32),
            pltpu.SemaphoreType.DMA,
            pltpu.SemaphoreType.DMA,
            pltpu.SemaphoreType.DMA,
            pltpu.SemaphoreType.DMA,
        ],
        compiler_params=pltpu.CompilerParams(
            use_tc_tiling_on_sc=True, needs_layout_passes=False
        ),
    )
    def gather_kernel(
        tab_hbm, idx_hbm, out_hbm, row_all, par_all, rows_v0, rows_v1,
        out_v0, out_v1, sem0, sem1, semo0, semo1
    ):
        wid = lax.axis_index("s") * 2 + lax.axis_index("c")
        rows_vs = (rows_v0, rows_v1)
        out_vs = (out_v0, out_v1)
        sems = (sem0, sem1)
        semos = (semo0, semo1)

        base = pl.multiple_of(wid * per_idx, 128)
        pltpu.sync_copy(idx_hbm.at[pl.ds(base, per_idx)], row_all)

        @plsc.parallel_loop(0, per_idx // _LANES, unroll=4)
        def _(v):
            sl = pl.ds(v * _LANES, _LANES)
            iv = row_all[sl]
            hi = iv >= n_half
            row_all[sl] = jnp.where(hi, iv - n_half, iv)
            par_all[sl] = jnp.where(hi, dim, 0).astype(jnp.int32)

        def gather_slice(s):
            return row_all.at[pl.ds(pl.multiple_of(s * 128, 128), 128)]

        def start(s, b):
            pltpu.async_copy(tab_hbm.at[gather_slice(s)], rows_vs[b], sems[b])

        def wait(s, b):
            pltpu.make_async_copy(
                tab_hbm.at[gather_slice(s)], rows_vs[b], sems[b]
            ).wait()

        def out_slice(s):
            g = wid * per_w + s
            b1 = g // n_tb
            tb = g % n_tb
            return out_hbm.at[b1, :, pl.ds(pl.multiple_of(tb * 128, 128), 128)]

        def wait_out(s, b):
            pltpu.make_async_copy(out_vs[b], out_slice(s), semos[b]).wait()

        start(0, 0)

        @pl.loop(0, per_w, step=2)
        def _(t):
            for b in range(2):
                s = t + b

                @pl.when(s + 1 < per_w)
                def _():
                    start(s + 1, 1 - b)

                wait(s, b)

                @pl.when(s >= 2)
                def _():
                    wait_out(s - 2, b)

                rows_b = rows_vs[b]
                out_b = out_vs[b]
                row16s = [rv * _LANES + _iota16() for rv in range(8)]
                par16s = [
                    par_all[pl.ds(s * 128 + rv * _LANES, _LANES)] for rv in range(8)
                ]

                @plsc.parallel_loop(0, dim, unroll=8)
                def _(f):
                    for rv in range(8):
                        val = plsc.load_gather(rows_b, [row16s[rv], par16s[rv] + f])
                        out_b[f, pl.ds(rv * _LANES, _LANES)] = val

                pltpu.async_copy(out_vs[b], out_slice(s), semos[b])

        wait_out(per_w - 2, 0)
        wait_out(per_w - 1, 1)

    return gather_kernel(packed, idx_flat_t)


def kernel(input, index):
    n_rows, dim = input.shape
    b0_n, b1_n = index.shape
    n_half = _NHI * _BLK

    table_t = input.T
    idx_flat_t = index.T.reshape(b1_n * b0_n)

    packed = _pack_table_tc(table_t, n_rows, dim)
    out_t = _gather_t(packed, idx_flat_t, n_half, b0_n, b1_n, dim)
    return out_t.transpose(2, 0, 1)
